# baseline (speedup 1.0000x reference)
import functools

import jax
import jax.numpy as jnp
from jax import lax
from jax.experimental import pallas as pl
from jax.experimental.pallas import tpu as pltpu
from jax.experimental.pallas import tpu_sc as plsc

_N, _E, _D = 10000, 320000, 128
_NC, _NS = 2, 16
_NW = _NC * _NS
_CH = 125
_CPB = 16
_NB = 5
_BLK = _CPB * _CH
_STRIPE = 624
_REM_OFF = _STRIPE * _NS
_REM = _N - _REM_OFF
_ZCH = 104


def _sc_scatter_body(x_hbm, ei_hbm, out_hbm,
                     sa, sb, da, db, rows, acc_sh, g0, g1, c0s, c1s, ssem):
    c = lax.axis_index("c")
    s = lax.axis_index("s")
    wid = c * _NS + s

    def stage(b, sblk, dblk):
        pltpu.async_copy(ei_hbm.at[0, wid, b], sblk, ssem)
        pltpu.async_copy(ei_hbm.at[1, wid, b], dblk, ssem)

    def stage_wait(sblk, dblk):
        pltpu.make_async_copy(ei_hbm.at[0, 0, 0], sblk, ssem).wait()
        pltpu.make_async_copy(ei_hbm.at[1, 0, 0], dblk, ssem).wait()

    def g_issue(idx_row, buf, sem):
        pltpu.async_copy(x_hbm.at[idx_row], buf, sem)

    def g_wait(buf, sem):
        pltpu.make_async_copy(x_hbm.at[sa.at[0]], buf, sem).wait()

    def s_issue(buf, idx_row, sem):
        pltpu.async_copy(buf, acc_sh.at[idx_row], sem, add=True)

    def s_wait(buf, sem):
        pltpu.make_async_copy(buf, acc_sh.at[da.at[0]], sem).wait()

    stage(0, sa, da)
    pltpu.make_async_copy(ei_hbm.at[0, 0, 0], sa, ssem).wait()
    g_issue(sa.at[0], rows.at[0], g0)

    def zrow(r, carry):
        for j in range(_D // 16):
            rows[1, r, pl.ds(j * 16, 16)] = jnp.zeros((16,), jnp.float32)
        return carry

    lax.fori_loop(0, _ZCH, zrow, 0)
    for k in range(_STRIPE // _ZCH):
        pltpu.async_copy(rows.at[1].at[pl.ds(0, _ZCH)],
                         acc_sh.at[pl.ds(s * _STRIPE + k * _ZCH, _ZCH)], c1s)

    @pl.when(s == _NS - 1)
    def _():
        pltpu.async_copy(rows.at[1].at[pl.ds(0, _REM)],
                         acc_sh.at[pl.ds(_REM_OFF, _REM)], c1s)

    pltpu.make_async_copy(ei_hbm.at[1, 0, 0], da, ssem).wait()
    stage(1, sb, db)
    for k in range(_STRIPE // _ZCH):
        pltpu.make_async_copy(rows.at[1].at[pl.ds(0, _ZCH)],
                              acc_sh.at[pl.ds(0, _ZCH)], c1s).wait()

    @pl.when(s == _NS - 1)
    def _():
        pltpu.make_async_copy(rows.at[1].at[pl.ds(0, _REM)],
                              acc_sh.at[pl.ds(0, _REM)], c1s).wait()

    plsc.subcore_barrier()

    def idx_row(blk, j):
        return blk.at[j]

    def steady_pair(sblk, dblk, j0):
        g_wait(rows.at[0], g0)
        s_issue(rows.at[0], idx_row(dblk, j0), c0s)
        s_wait(rows.at[1], c1s)
        g_issue(idx_row(sblk, j0 + 1), rows.at[1], g1)
        s_wait(rows.at[0], c0s)
        g_issue(idx_row(sblk, j0 + 2), rows.at[0], g0)
        g_wait(rows.at[1], g1)
        s_issue(rows.at[1], idx_row(dblk, j0 + 1), c1s)

    def process_block(b, sblk, dblk, so, do_):
        g_wait(rows.at[0], g0)
        s_issue(rows.at[0], idx_row(dblk, 0), c0s)
        if b > 0:
            s_wait(rows.at[1], c1s)
            if b + 1 < _NB:
                stage(b + 1, so, do_)
        g_issue(idx_row(sblk, 1), rows.at[1], g1)
        s_wait(rows.at[0], c0s)
        g_issue(idx_row(sblk, 2), rows.at[0], g0)
        g_wait(rows.at[1], g1)
        s_issue(rows.at[1], idx_row(dblk, 1), c1s)

        def pair(i, carry):
            steady_pair(sblk, dblk, 2 * i)
            return carry

        lax.fori_loop(1, _CPB // 2 - 1, pair, 0)

        j0 = _CPB - 2
        g_wait(rows.at[0], g0)
        s_issue(rows.at[0], idx_row(dblk, j0), c0s)
        s_wait(rows.at[1], c1s)
        g_issue(idx_row(sblk, j0 + 1), rows.at[1], g1)
        s_wait(rows.at[0], c0s)
        if b + 1 < _NB:
            stage_wait(so, do_)
            g_issue(idx_row(so, 0), rows.at[0], g0)
        g_wait(rows.at[1], g1)
        s_issue(rows.at[1], idx_row(dblk, j0 + 1), c1s)
        if b + 1 == _NB:
            s_wait(rows.at[1], c1s)

    for b in range(_NB):
        sblk, dblk, so, do_ = (sa, da, sb, db) if b % 2 == 0 else (sb, db, sa, da)
        process_block(b, sblk, dblk, so, do_)

    plsc.subcore_barrier()
    pltpu.sync_copy(acc_sh.at[pl.ds(s * _STRIPE, _STRIPE)],
                    out_hbm.at[pl.ds(c * _N + s * _STRIPE, _STRIPE)])

    @pl.when(s == _NS - 1)
    def _():
        pltpu.sync_copy(acc_sh.at[pl.ds(_REM_OFF, _REM)],
                        out_hbm.at[pl.ds(c * _N + _REM_OFF, _REM)])


_sc_scatter = pl.kernel(
    _sc_scatter_body,
    out_type=jax.ShapeDtypeStruct((_NC * _N, _D), jnp.float32),
    mesh=plsc.VectorSubcoreMesh(core_axis_name="c", subcore_axis_name="s"),
    scratch_types=[
        pltpu.VMEM((_CPB, _CH), jnp.int32),
        pltpu.VMEM((_CPB, _CH), jnp.int32),
        pltpu.VMEM((_CPB, _CH), jnp.int32),
        pltpu.VMEM((_CPB, _CH), jnp.int32),
        pltpu.VMEM((2, _CH, _D), jnp.float32),
        pltpu.VMEM_SHARED((_N, _D), jnp.float32),
        pltpu.SemaphoreType.DMA,
        pltpu.SemaphoreType.DMA,
        pltpu.SemaphoreType.DMA,
        pltpu.SemaphoreType.DMA,
        pltpu.SemaphoreType.DMA,
    ],
)


def _mlp_core(z, x_res, w1t_ref, b1_ref, g1_ref, be1_ref,
              w2t_ref, b2_ref, g2_ref, be2_ref):
    t = jnp.dot(z, w1t_ref[...], preferred_element_type=jnp.float32)
    t = t + b1_ref[...]
    m = jnp.mean(t, axis=0, keepdims=True)
    v = jnp.mean((t - m) * (t - m), axis=0, keepdims=True)
    t = (t - m) / jnp.sqrt(v + 1e-5) * g1_ref[...] + be1_ref[...]
    t = jnp.maximum(t, 0.0)
    u = jnp.dot(t, w2t_ref[...], preferred_element_type=jnp.float32)
    u = u + b2_ref[...]
    m2 = jnp.mean(u, axis=0, keepdims=True)
    v2 = jnp.mean((u - m2) * (u - m2), axis=0, keepdims=True)
    u = (u - m2) / jnp.sqrt(v2 + 1e-5) * g2_ref[...] + be2_ref[...]
    return jnp.maximum(u, 0.0) + x_res


def _mlp0_body(p_ref, x_ref, w1t_ref, b1_ref, g1_ref, be1_ref,
               w2t_ref, b2_ref, g2_ref, be2_ref, o_ref):
    x = x_ref[...]
    z = x + p_ref[0:_N, :] + p_ref[_N:2 * _N, :]
    o_ref[...] = _mlp_core(z, x, w1t_ref, b1_ref, g1_ref, be1_ref,
                           w2t_ref, b2_ref, g2_ref, be2_ref)


def _mlp1_body(h_ref, p_ref, x_ref, w1t_ref, b1_ref, g1_ref, be1_ref,
               w2t_ref, b2_ref, g2_ref, be2_ref, o_ref):
    z = h_ref[...] + p_ref[0:_N, :] + p_ref[_N:2 * _N, :]
    o_ref[...] = _mlp_core(z, x_ref[...], w1t_ref, b1_ref, g1_ref, be1_ref,
                           w2t_ref, b2_ref, g2_ref, be2_ref)


_mlp0_call = pl.pallas_call(
    _mlp0_body, out_shape=jax.ShapeDtypeStruct((_N, _D), jnp.float32))
_mlp1_call = pl.pallas_call(
    _mlp1_body, out_shape=jax.ShapeDtypeStruct((_N, _D), jnp.float32))


def kernel(x, edge_index,
           l0_W1, l0_b1, l0_g1, l0_be1, l0_W2, l0_b2, l0_g2, l0_be2,
           l1_W1, l1_b1, l1_g1, l1_be1, l1_W2, l1_b2, l1_g2, l1_be2):
    ei = edge_index.astype(jnp.int32).reshape(2, _NW, _NB, _CPB, _CH)

    def wparams(W1, b1, g1, be1, W2, b2, g2, be2):
        return (W1.T, b1.reshape(1, _D), g1.reshape(1, _D),
                be1.reshape(1, _D),
                W2.T, b2.reshape(1, _D), g2.reshape(1, _D),
                be2.reshape(1, _D))

    p0 = _sc_scatter(x, ei)
    h = _mlp0_call(p0, x, *wparams(l0_W1, l0_b1, l0_g1, l0_be1,
                                   l0_W2, l0_b2, l0_g2, l0_be2))
    p1 = _sc_scatter(h, ei)
    return _mlp1_call(h, p1, x, *wparams(l1_W1, l1_b1, l1_g1, l1_be1,
                                         l1_W2, l1_b2, l1_g2, l1_be2))


# speedup vs baseline: 1.0578x; 1.0578x over previous
import functools

import jax
import jax.numpy as jnp
from jax import lax
from jax.experimental import pallas as pl
from jax.experimental.pallas import tpu as pltpu
from jax.experimental.pallas import tpu_sc as plsc

_N, _E, _D = 10000, 320000, 128
_NC, _NS = 2, 16
_NW = _NC * _NS
_CH = 50
_CPB = 40
_NB = 5
_STRIPE = 624
_REM_OFF = _STRIPE * _NS
_REM = _N - _REM_OFF
_ZCH = 48


def _sc_scatter_body(x_hbm, ei_hbm, out_hbm, sa, sb, da, db, rows, acc_sh,
                     g0, g1, g2, g3, c0, c1, c2, c3, ssem):
    c = lax.axis_index("c")
    s = lax.axis_index("s")
    wid = c * _NS + s
    gss = [g0, g1, g2, g3]
    css = [c0, c1, c2, c3]

    def stage(b, sblk, dblk):
        pltpu.async_copy(ei_hbm.at[0, wid, b], sblk, ssem)
        pltpu.async_copy(ei_hbm.at[1, wid, b], dblk, ssem)

    def stage_wait(sblk, dblk):
        pltpu.make_async_copy(ei_hbm.at[0, 0, 0], sblk, ssem).wait()
        pltpu.make_async_copy(ei_hbm.at[1, 0, 0], dblk, ssem).wait()

    def g_issue(idx_row, buf, sem):
        pltpu.async_copy(x_hbm.at[idx_row], buf, sem)

    def g_wait(buf, sem):
        pltpu.make_async_copy(x_hbm.at[sa.at[0]], buf, sem).wait()

    def s_issue(buf, idx_row, sem):
        pltpu.async_copy(buf, acc_sh.at[idx_row], sem, add=True)

    def s_wait(buf, sem):
        pltpu.make_async_copy(buf, acc_sh.at[da.at[0]], sem).wait()

    stage(0, sa, da)
    pltpu.make_async_copy(ei_hbm.at[0, 0, 0], sa, ssem).wait()
    g_issue(sa.at[0], rows.at[0], g0)
    g_issue(sa.at[1], rows.at[1], g1)

    def zrow(r, carry):
        for j in range(_D // 16):
            rows[2, r, pl.ds(j * 16, 16)] = jnp.zeros((16,), jnp.float32)
        return carry

    lax.fori_loop(0, _ZCH, zrow, 0)
    for k in range(_STRIPE // _ZCH):
        pltpu.async_copy(rows.at[2].at[pl.ds(0, _ZCH)],
                         acc_sh.at[pl.ds(s * _STRIPE + k * _ZCH, _ZCH)], c2)

    @pl.when(s == _NS - 1)
    def _():
        pltpu.async_copy(rows.at[2].at[pl.ds(0, _REM)],
                         acc_sh.at[pl.ds(_REM_OFF, _REM)], c2)

    pltpu.make_async_copy(ei_hbm.at[1, 0, 0], da, ssem).wait()
    stage(1, sb, db)
    for k in range(_STRIPE // _ZCH):
        pltpu.make_async_copy(rows.at[2].at[pl.ds(0, _ZCH)],
                              acc_sh.at[pl.ds(0, _ZCH)], c2).wait()

    @pl.when(s == _NS - 1)
    def _():
        pltpu.make_async_copy(rows.at[2].at[pl.ds(0, _REM)],
                              acc_sh.at[pl.ds(0, _REM)], c2).wait()

    plsc.subcore_barrier()

    def chunk(sblk, dblk, k, ph, skip_swait=False, pf="std"):
        m = ph % 4
        f = (ph + 2) % 4
        g_wait(rows.at[m], gss[m])
        if not skip_swait:
            s_wait(rows.at[f], css[f])
        if pf == "std":
            g_issue(sblk.at[k + 2], rows.at[f], gss[f])
        elif pf is not None:
            g_issue(pf[0].at[pf[1]], rows.at[f], gss[f])
        s_issue(rows.at[m], dblk.at[k], css[m])

    def process_block(b, sblk, dblk, so, do_):
        chunk(sblk, dblk, 0, 0, skip_swait=(b == 0))
        chunk(sblk, dblk, 1, 1, skip_swait=(b == 0))
        if 0 < b < _NB - 1:
            stage(b + 1, so, do_)

        def quad(i, carry):
            k = 4 * i + 2
            chunk(sblk, dblk, k, 2)
            chunk(sblk, dblk, k + 1, 3)
            chunk(sblk, dblk, k + 2, 0)
            chunk(sblk, dblk, k + 3, 1)
            return carry

        lax.fori_loop(0, (_CPB - 4) // 4, quad, 0)

        if b + 1 < _NB:
            g_wait(rows.at[2], g2)
            s_wait(rows.at[0], c0)
            stage_wait(so, do_)
            g_issue(so.at[0], rows.at[0], g0)
            s_issue(rows.at[2], dblk.at[_CPB - 2], c2)
            g_wait(rows.at[3], g3)
            s_wait(rows.at[1], c1)
            g_issue(so.at[1], rows.at[1], g1)
            s_issue(rows.at[3], dblk.at[_CPB - 1], c3)
        else:
            chunk(sblk, dblk, _CPB - 2, 2, pf=None)
            chunk(sblk, dblk, _CPB - 1, 3, pf=None)
            s_wait(rows.at[2], c2)
            s_wait(rows.at[3], c3)

    for b in range(_NB):
        sblk, dblk, so, do_ = (sa, da, sb, db) if b % 2 == 0 else (sb, db, sa, da)
        process_block(b, sblk, dblk, so, do_)

    plsc.subcore_barrier()
    pltpu.sync_copy(acc_sh.at[pl.ds(s * _STRIPE, _STRIPE)],
                    out_hbm.at[pl.ds(c * _N + s * _STRIPE, _STRIPE)])

    @pl.when(s == _NS - 1)
    def _():
        pltpu.sync_copy(acc_sh.at[pl.ds(_REM_OFF, _REM)],
                        out_hbm.at[pl.ds(c * _N + _REM_OFF, _REM)])


_sc_scatter = pl.kernel(
    _sc_scatter_body,
    out_type=jax.ShapeDtypeStruct((_NC * _N, _D), jnp.float32),
    mesh=plsc.VectorSubcoreMesh(core_axis_name="c", subcore_axis_name="s"),
    scratch_types=[
        pltpu.VMEM((_CPB, _CH), jnp.int32),
        pltpu.VMEM((_CPB, _CH), jnp.int32),
        pltpu.VMEM((_CPB, _CH), jnp.int32),
        pltpu.VMEM((_CPB, _CH), jnp.int32),
        pltpu.VMEM((4, _CH, _D), jnp.float32),
        pltpu.VMEM_SHARED((_N, _D), jnp.float32),
        pltpu.SemaphoreType.DMA,
        pltpu.SemaphoreType.DMA,
        pltpu.SemaphoreType.DMA,
        pltpu.SemaphoreType.DMA,
        pltpu.SemaphoreType.DMA,
        pltpu.SemaphoreType.DMA,
        pltpu.SemaphoreType.DMA,
        pltpu.SemaphoreType.DMA,
        pltpu.SemaphoreType.DMA,
    ],
)


def _mlp_core(z, x_res, w1t_ref, b1_ref, g1_ref, be1_ref,
              w2t_ref, b2_ref, g2_ref, be2_ref):
    t = jnp.dot(z, w1t_ref[...], preferred_element_type=jnp.float32)
    t = t + b1_ref[...]
    m = jnp.mean(t, axis=0, keepdims=True)
    v = jnp.mean((t - m) * (t - m), axis=0, keepdims=True)
    t = (t - m) / jnp.sqrt(v + 1e-5) * g1_ref[...] + be1_ref[...]
    t = jnp.maximum(t, 0.0)
    u = jnp.dot(t, w2t_ref[...], preferred_element_type=jnp.float32)
    u = u + b2_ref[...]
    m2 = jnp.mean(u, axis=0, keepdims=True)
    v2 = jnp.mean((u - m2) * (u - m2), axis=0, keepdims=True)
    u = (u - m2) / jnp.sqrt(v2 + 1e-5) * g2_ref[...] + be2_ref[...]
    return jnp.maximum(u, 0.0) + x_res


def _mlp0_body(p_ref, x_ref, w1t_ref, b1_ref, g1_ref, be1_ref,
               w2t_ref, b2_ref, g2_ref, be2_ref, o_ref):
    x = x_ref[...]
    z = x + p_ref[0:_N, :] + p_ref[_N:2 * _N, :]
    o_ref[...] = _mlp_core(z, x, w1t_ref, b1_ref, g1_ref, be1_ref,
                           w2t_ref, b2_ref, g2_ref, be2_ref)


def _mlp1_body(h_ref, p_ref, x_ref, w1t_ref, b1_ref, g1_ref, be1_ref,
               w2t_ref, b2_ref, g2_ref, be2_ref, o_ref):
    z = h_ref[...] + p_ref[0:_N, :] + p_ref[_N:2 * _N, :]
    o_ref[...] = _mlp_core(z, x_ref[...], w1t_ref, b1_ref, g1_ref, be1_ref,
                           w2t_ref, b2_ref, g2_ref, be2_ref)


_mlp0_call = pl.pallas_call(
    _mlp0_body, out_shape=jax.ShapeDtypeStruct((_N, _D), jnp.float32))
_mlp1_call = pl.pallas_call(
    _mlp1_body, out_shape=jax.ShapeDtypeStruct((_N, _D), jnp.float32))


def kernel(x, edge_index,
           l0_W1, l0_b1, l0_g1, l0_be1, l0_W2, l0_b2, l0_g2, l0_be2,
           l1_W1, l1_b1, l1_g1, l1_be1, l1_W2, l1_b2, l1_g2, l1_be2):
    ei = edge_index.astype(jnp.int32).reshape(2, _NW, _NB, _CPB, _CH)

    def wparams(W1, b1, g1, be1, W2, b2, g2, be2):
        return (W1.T, b1.reshape(1, _D), g1.reshape(1, _D),
                be1.reshape(1, _D),
                W2.T, b2.reshape(1, _D), g2.reshape(1, _D),
                be2.reshape(1, _D))

    p0 = _sc_scatter(x, ei)
    h = _mlp0_call(p0, x, *wparams(l0_W1, l0_b1, l0_g1, l0_be1,
                                   l0_W2, l0_b2, l0_g2, l0_be2))
    p1 = _sc_scatter(h, ei)
    return _mlp1_call(h, p1, x, *wparams(l1_W1, l1_b1, l1_g1, l1_be1,
                                         l1_W2, l1_b2, l1_g2, l1_be2))


# speedup vs baseline: 1.0660x; 1.0078x over previous
import functools

import jax
import jax.numpy as jnp
from jax import lax
from jax.experimental import pallas as pl
from jax.experimental.pallas import tpu as pltpu
from jax.experimental.pallas import tpu_sc as plsc

_N, _E, _D = 10000, 320000, 128
_NC, _NS = 2, 16
_NW = _NC * _NS
_CH = 125
_CPB = 16
_NB = 5
_BLK = _CPB * _CH
_STRIPE = 624
_REM_OFF = _STRIPE * _NS
_REM = _N - _REM_OFF
_ZCH = 104


def _sc_scatter_body(x_hbm, ei_hbm, out_hbm,
                     sa, sb, da, db, rows, acc_sh, g0, g1, c0s, c1s, ssem):
    c = lax.axis_index("c")
    s = lax.axis_index("s")
    wid = c * _NS + s

    def stage(b, sblk, dblk):
        pltpu.async_copy(ei_hbm.at[0, wid, b], sblk, ssem)
        pltpu.async_copy(ei_hbm.at[1, wid, b], dblk, ssem)

    def stage_wait(sblk, dblk):
        pltpu.make_async_copy(ei_hbm.at[0, 0, 0], sblk, ssem).wait()
        pltpu.make_async_copy(ei_hbm.at[1, 0, 0], dblk, ssem).wait()

    def g_issue(idx_row, buf, sem):
        pltpu.async_copy(x_hbm.at[idx_row], buf, sem)

    def g_wait(buf, sem):
        pltpu.make_async_copy(x_hbm.at[sa.at[0]], buf, sem).wait()

    def s_issue(buf, idx_row, sem):
        pltpu.async_copy(buf, acc_sh.at[idx_row], sem, add=True)

    def s_wait(buf, sem):
        pltpu.make_async_copy(buf, acc_sh.at[da.at[0]], sem).wait()

    stage(0, sa, da)
    pltpu.make_async_copy(ei_hbm.at[0, 0, 0], sa, ssem).wait()
    g_issue(sa.at[0], rows.at[0], g0)

    def zrow(r, carry):
        for j in range(_D // 16):
            rows[1, r, pl.ds(j * 16, 16)] = jnp.zeros((16,), jnp.float32)
        return carry

    lax.fori_loop(0, _ZCH, zrow, 0)
    for k in range(_STRIPE // _ZCH):
        pltpu.async_copy(rows.at[1].at[pl.ds(0, _ZCH)],
                         acc_sh.at[pl.ds(s * _STRIPE + k * _ZCH, _ZCH)], c1s)

    @pl.when(s == _NS - 1)
    def _():
        pltpu.async_copy(rows.at[1].at[pl.ds(0, _REM)],
                         acc_sh.at[pl.ds(_REM_OFF, _REM)], c1s)

    pltpu.make_async_copy(ei_hbm.at[1, 0, 0], da, ssem).wait()
    stage(1, sb, db)
    for k in range(_STRIPE // _ZCH):
        pltpu.make_async_copy(rows.at[1].at[pl.ds(0, _ZCH)],
                              acc_sh.at[pl.ds(0, _ZCH)], c1s).wait()

    @pl.when(s == _NS - 1)
    def _():
        pltpu.make_async_copy(rows.at[1].at[pl.ds(0, _REM)],
                              acc_sh.at[pl.ds(0, _REM)], c1s).wait()

    plsc.subcore_barrier()

    def idx_row(blk, j):
        return blk.at[j]

    def steady_pair(sblk, dblk, j0):
        g_wait(rows.at[0], g0)
        s_issue(rows.at[0], idx_row(dblk, j0), c0s)
        s_wait(rows.at[1], c1s)
        g_issue(idx_row(sblk, j0 + 1), rows.at[1], g1)
        g_wait(rows.at[1], g1)
        s_issue(rows.at[1], idx_row(dblk, j0 + 1), c1s)
        s_wait(rows.at[0], c0s)
        g_issue(idx_row(sblk, j0 + 2), rows.at[0], g0)

    def process_block(b, sblk, dblk, so, do_):
        g_wait(rows.at[0], g0)
        s_issue(rows.at[0], idx_row(dblk, 0), c0s)
        if b > 0:
            s_wait(rows.at[1], c1s)
            if b + 1 < _NB:
                stage(b + 1, so, do_)
        g_issue(idx_row(sblk, 1), rows.at[1], g1)
        g_wait(rows.at[1], g1)
        s_issue(rows.at[1], idx_row(dblk, 1), c1s)
        s_wait(rows.at[0], c0s)
        g_issue(idx_row(sblk, 2), rows.at[0], g0)

        def pair(i, carry):
            steady_pair(sblk, dblk, 2 * i)
            return carry

        lax.fori_loop(1, _CPB // 2 - 1, pair, 0)

        j0 = _CPB - 2
        g_wait(rows.at[0], g0)
        s_issue(rows.at[0], idx_row(dblk, j0), c0s)
        s_wait(rows.at[1], c1s)
        g_issue(idx_row(sblk, j0 + 1), rows.at[1], g1)
        g_wait(rows.at[1], g1)
        s_issue(rows.at[1], idx_row(dblk, j0 + 1), c1s)
        s_wait(rows.at[0], c0s)
        if b + 1 < _NB:
            stage_wait(so, do_)
            g_issue(idx_row(so, 0), rows.at[0], g0)
        else:
            s_wait(rows.at[1], c1s)

    for b in range(_NB):
        sblk, dblk, so, do_ = (sa, da, sb, db) if b % 2 == 0 else (sb, db, sa, da)
        process_block(b, sblk, dblk, so, do_)

    plsc.subcore_barrier()
    pltpu.sync_copy(acc_sh.at[pl.ds(s * _STRIPE, _STRIPE)],
                    out_hbm.at[pl.ds(c * _N + s * _STRIPE, _STRIPE)])

    @pl.when(s == _NS - 1)
    def _():
        pltpu.sync_copy(acc_sh.at[pl.ds(_REM_OFF, _REM)],
                        out_hbm.at[pl.ds(c * _N + _REM_OFF, _REM)])


_sc_scatter = pl.kernel(
    _sc_scatter_body,
    out_type=jax.ShapeDtypeStruct((_NC * _N, _D), jnp.float32),
    mesh=plsc.VectorSubcoreMesh(core_axis_name="c", subcore_axis_name="s"),
    scratch_types=[
        pltpu.VMEM((_CPB, _CH), jnp.int32),
        pltpu.VMEM((_CPB, _CH), jnp.int32),
        pltpu.VMEM((_CPB, _CH), jnp.int32),
        pltpu.VMEM((_CPB, _CH), jnp.int32),
        pltpu.VMEM((2, _CH, _D), jnp.float32),
        pltpu.VMEM_SHARED((_N, _D), jnp.float32),
        pltpu.SemaphoreType.DMA,
        pltpu.SemaphoreType.DMA,
        pltpu.SemaphoreType.DMA,
        pltpu.SemaphoreType.DMA,
        pltpu.SemaphoreType.DMA,
    ],
)


def _mlp_core(z, x_res, w1t_ref, b1_ref, g1_ref, be1_ref,
              w2t_ref, b2_ref, g2_ref, be2_ref):
    t = jnp.dot(z, w1t_ref[...], preferred_element_type=jnp.float32)
    t = t + b1_ref[...]
    m = jnp.mean(t, axis=0, keepdims=True)
    v = jnp.mean((t - m) * (t - m), axis=0, keepdims=True)
    t = (t - m) / jnp.sqrt(v + 1e-5) * g1_ref[...] + be1_ref[...]
    t = jnp.maximum(t, 0.0)
    u = jnp.dot(t, w2t_ref[...], preferred_element_type=jnp.float32)
    u = u + b2_ref[...]
    m2 = jnp.mean(u, axis=0, keepdims=True)
    v2 = jnp.mean((u - m2) * (u - m2), axis=0, keepdims=True)
    u = (u - m2) / jnp.sqrt(v2 + 1e-5) * g2_ref[...] + be2_ref[...]
    return jnp.maximum(u, 0.0) + x_res


def _mlp0_body(p_ref, x_ref, w1t_ref, b1_ref, g1_ref, be1_ref,
               w2t_ref, b2_ref, g2_ref, be2_ref, o_ref):
    x = x_ref[...]
    z = x + p_ref[0:_N, :] + p_ref[_N:2 * _N, :]
    o_ref[...] = _mlp_core(z, x, w1t_ref, b1_ref, g1_ref, be1_ref,
                           w2t_ref, b2_ref, g2_ref, be2_ref)


def _mlp1_body(h_ref, p_ref, x_ref, w1t_ref, b1_ref, g1_ref, be1_ref,
               w2t_ref, b2_ref, g2_ref, be2_ref, o_ref):
    z = h_ref[...] + p_ref[0:_N, :] + p_ref[_N:2 * _N, :]
    o_ref[...] = _mlp_core(z, x_ref[...], w1t_ref, b1_ref, g1_ref, be1_ref,
                           w2t_ref, b2_ref, g2_ref, be2_ref)


_mlp0_call = pl.pallas_call(
    _mlp0_body, out_shape=jax.ShapeDtypeStruct((_N, _D), jnp.float32))
_mlp1_call = pl.pallas_call(
    _mlp1_body, out_shape=jax.ShapeDtypeStruct((_N, _D), jnp.float32))


def kernel(x, edge_index,
           l0_W1, l0_b1, l0_g1, l0_be1, l0_W2, l0_b2, l0_g2, l0_be2,
           l1_W1, l1_b1, l1_g1, l1_be1, l1_W2, l1_b2, l1_g2, l1_be2):
    ei = edge_index.astype(jnp.int32).reshape(2, _NW, _NB, _CPB, _CH)

    def wparams(W1, b1, g1, be1, W2, b2, g2, be2):
        return (W1.T, b1.reshape(1, _D), g1.reshape(1, _D),
                be1.reshape(1, _D),
                W2.T, b2.reshape(1, _D), g2.reshape(1, _D),
                be2.reshape(1, _D))

    p0 = _sc_scatter(x, ei)
    h = _mlp0_call(p0, x, *wparams(l0_W1, l0_b1, l0_g1, l0_be1,
                                   l0_W2, l0_b2, l0_g2, l0_be2))
    p1 = _sc_scatter(h, ei)
    return _mlp1_call(h, p1, x, *wparams(l1_W1, l1_b1, l1_g1, l1_be1,
                                         l1_W2, l1_b2, l1_g2, l1_be2))
